# baseline (speedup 1.0000x reference)
import functools

import jax
import jax.numpy as jnp
from jax import lax
from jax.experimental import pallas as pl
from jax.experimental.pallas import tpu as pltpu
from jax.experimental.pallas import tpu_sc as plsc

_N = 10000
_M = 16
_AF = 128
_NCONV = 3
_NCRYS = 200
_APC = _N // _NCRYS
_NBF = 41
_NBF_P = 48
_INV_VAR = 1.0 / (0.2 * 0.2)

_E = _N * _M

_SC_NC = 2
_SC_NS = 16
_NW = _SC_NC * _SC_NS
_CH = 125
_NCHUNK = _E // _CH
_CPW = _NCHUNK // _NW

_BN = 1000
_GRID = _N // _BN


def _sc_gather(x, idx2d):
    dt = x.dtype
    width = x.shape[1]
    mesh = plsc.VectorSubcoreMesh(core_axis_name="c", subcore_axis_name="s")

    @functools.partial(
        pl.kernel,
        mesh=mesh,
        out_type=jax.ShapeDtypeStruct((_E, width), dt),
        compiler_params=pltpu.CompilerParams(use_tc_tiling_on_sc=False),
        scratch_types=[
            pltpu.VMEM((_CPW, _CH), jnp.int32),
            pltpu.VMEM((_CH, width), dt),
            pltpu.VMEM((_CH, width), dt),
            pltpu.SemaphoreType.DMA,
            pltpu.SemaphoreType.DMA,
        ],
    )
    def gath(x_hbm, idx_hbm, out_hbm, idx_v, rows0, rows1, sem0, sem1):
        wid = lax.axis_index("s") * _SC_NC + lax.axis_index("c")
        start = wid * _CPW
        pltpu.sync_copy(idx_hbm.at[pl.ds(start, _CPW)], idx_v)
        bufs = (rows0, rows1)
        sems = (sem0, sem1)
        pltpu.async_copy(x_hbm.at[idx_v.at[0]], rows0, sem0)

        def body(k, carry):
            for b in range(2):
                c = k * 2 + b
                nb = 1 - b

                @pl.when(c + 1 < _CPW)
                def _pref():
                    pltpu.async_copy(x_hbm.at[idx_v.at[c + 1]], bufs[nb],
                                     sems[nb])

                pltpu.make_async_copy(x_hbm.at[idx_v.at[c]], bufs[b],
                                      sems[b]).wait()
                pltpu.sync_copy(bufs[b],
                                out_hbm.at[pl.ds((start + c) * _CH, _CH)])
            return carry

        lax.fori_loop(0, _CPW // 2, body, 0)

    return gath(x, idx2d)


def _softplus(v):
    return jnp.maximum(v, 0.0) + jnp.log(1.0 + jnp.exp(-jnp.abs(v)))


def _wide_specs():
    return [
        pl.BlockSpec((1, _BN, _AF), lambda i, m: (m, i, 0)),
        pl.BlockSpec((_BN, _AF), lambda i, m: (i, 0)),
        pl.BlockSpec((1, _BN, 1), lambda i, m: (m, i, 0)),
        pl.BlockSpec((_AF, 2 * _AF), lambda i, m: (0, 0)),
        pl.BlockSpec((_AF, 2 * _AF), lambda i, m: (0, 0)),
        pl.BlockSpec((_NBF_P, 2 * _AF), lambda i, m: (0, 0)),
        pl.BlockSpec((1, 2 * _AF), lambda i, m: (0, 0)),
        pl.BlockSpec((1, _NBF_P), lambda i, m: (0, 0)),
    ]


def _gated_m(xg_ref, nbr_ref, wn_ref, wg_ref, filt_ref, s):
    g = jnp.exp(-jnp.square(nbr_ref[0] - filt_ref[...]) * _INV_VAR)
    um = jnp.dot(xg_ref[0].astype(jnp.bfloat16), wn_ref[...],
                 preferred_element_type=jnp.float32)
    gm = jnp.dot(g.astype(wg_ref.dtype), wg_ref[...],
                 preferred_element_type=jnp.float32)
    return um + gm + s


def _pass1(xg, x, nbrT, ws, wn, wg, fcb, filt):
    def body(xg_ref, x_ref, nbr_ref, ws_ref, wn_ref, wg_ref,
             fcb_ref, filt_ref, sum_ref, sq_ref, s_ref):
        m = pl.program_id(1)

        @pl.when(m == 0)
        def _s():
            s_ref[...] = jnp.dot(x_ref[...], ws_ref[...],
                                 preferred_element_type=jnp.float32
                                 ) + fcb_ref[...]

        gated = _gated_m(xg_ref, nbr_ref, wn_ref, wg_ref, filt_ref, s_ref[...])

        @pl.when((pl.program_id(0) == 0) & (m == 0))
        def _init():
            sum_ref[...] = jnp.zeros_like(sum_ref)
            sq_ref[...] = jnp.zeros_like(sq_ref)

        sum_ref[...] += jnp.sum(gated, axis=0, keepdims=True)
        sq_ref[...] += jnp.sum(gated * gated, axis=0, keepdims=True)

    return pl.pallas_call(
        body,
        grid=(_GRID, _M),
        in_specs=_wide_specs(),
        out_specs=[
            pl.BlockSpec((1, 2 * _AF), lambda i, m: (0, 0)),
            pl.BlockSpec((1, 2 * _AF), lambda i, m: (0, 0)),
        ],
        out_shape=[
            jax.ShapeDtypeStruct((1, 2 * _AF), jnp.float32),
            jax.ShapeDtypeStruct((1, 2 * _AF), jnp.float32),
        ],
        scratch_shapes=[pltpu.VMEM((_BN, 2 * _AF), jnp.float32)],
        compiler_params=pltpu.CompilerParams(
            dimension_semantics=("arbitrary", "arbitrary")),
    )(xg, x, nbrT, ws, wn, wg, fcb, filt)


def _pass2(xg, x, nbrT, ws, wn, wg, fcb, filt, sum1, sq1, g1, b1):
    def body(xg_ref, x_ref, nbr_ref, ws_ref, wn_ref, wg_ref,
             fcb_ref, filt_ref, s1_ref, q1_ref, g1_ref, b1_ref,
             ns_ref, sum_ref, sq_ref, s_ref):
        m = pl.program_id(1)
        cnt = jnp.float32(_E)
        mu = s1_ref[...] / cnt
        var = q1_ref[...] / cnt - mu * mu
        inv = lax.rsqrt(var + 1e-5)
        scale = g1_ref[...] * inv
        shift = b1_ref[...] - mu * scale

        @pl.when(m == 0)
        def _s():
            s_ref[...] = jnp.dot(x_ref[...], ws_ref[...],
                                 preferred_element_type=jnp.float32
                                 ) + fcb_ref[...]

        gated = _gated_m(xg_ref, nbr_ref, wn_ref, wg_ref, filt_ref, s_ref[...])
        gh = gated * scale + shift
        f = 1.0 / (1.0 + jnp.exp(-gh[:, :_AF]))
        c = _softplus(gh[:, _AF:])
        contrib = f * c

        @pl.when(m == 0)
        def _first():
            ns_ref[...] = contrib

        @pl.when(m > 0)
        def _rest():
            ns_ref[...] += contrib

        @pl.when((pl.program_id(0) == 0) & (m == _M - 1))
        def _init():
            sum_ref[...] = jnp.zeros_like(sum_ref)
            sq_ref[...] = jnp.zeros_like(sq_ref)

        @pl.when(m == _M - 1)
        def _stats():
            a = ns_ref[...]
            sum_ref[...] += jnp.sum(a, axis=0, keepdims=True)
            sq_ref[...] += jnp.sum(a * a, axis=0, keepdims=True)

    return pl.pallas_call(
        body,
        grid=(_GRID, _M),
        in_specs=_wide_specs() + [
            pl.BlockSpec((1, 2 * _AF), lambda i, m: (0, 0)),
            pl.BlockSpec((1, 2 * _AF), lambda i, m: (0, 0)),
            pl.BlockSpec((1, 2 * _AF), lambda i, m: (0, 0)),
            pl.BlockSpec((1, 2 * _AF), lambda i, m: (0, 0)),
        ],
        out_specs=[
            pl.BlockSpec((_BN, _AF), lambda i, m: (i, 0)),
            pl.BlockSpec((1, _AF), lambda i, m: (0, 0)),
            pl.BlockSpec((1, _AF), lambda i, m: (0, 0)),
        ],
        out_shape=[
            jax.ShapeDtypeStruct((_N, _AF), jnp.float32),
            jax.ShapeDtypeStruct((1, _AF), jnp.float32),
            jax.ShapeDtypeStruct((1, _AF), jnp.float32),
        ],
        scratch_shapes=[pltpu.VMEM((_BN, 2 * _AF), jnp.float32)],
        compiler_params=pltpu.CompilerParams(
            dimension_semantics=("arbitrary", "arbitrary")),
    )(xg, x, nbrT, ws, wn, wg, fcb, filt, sum1, sq1, g1, b1)


def _bn2_apply(x_ref, ns_ref, s2_ref, q2_ref, g2_ref, b2_ref):
    cnt = jnp.float32(_N)
    mu = s2_ref[...] / cnt
    var = q2_ref[...] / cnt - mu * mu
    inv = lax.rsqrt(var + 1e-5)
    scale = g2_ref[...] * inv
    shift = b2_ref[...] - mu * scale
    return _softplus(x_ref[...] + ns_ref[...] * scale + shift)


def _pass3(x, ns, sum2, sq2, g2, b2):
    def body(x_ref, ns_ref, s2_ref, q2_ref, g2_ref, b2_ref, out_ref):
        out_ref[...] = _bn2_apply(x_ref, ns_ref, s2_ref, q2_ref, g2_ref, b2_ref)

    return pl.pallas_call(
        body,
        grid=(_GRID,),
        in_specs=[
            pl.BlockSpec((_BN, _AF), lambda i: (i, 0)),
            pl.BlockSpec((_BN, _AF), lambda i: (i, 0)),
            pl.BlockSpec((1, _AF), lambda i: (0, 0)),
            pl.BlockSpec((1, _AF), lambda i: (0, 0)),
            pl.BlockSpec((1, _AF), lambda i: (0, 0)),
            pl.BlockSpec((1, _AF), lambda i: (0, 0)),
        ],
        out_specs=pl.BlockSpec((_BN, _AF), lambda i: (i, 0)),
        out_shape=jax.ShapeDtypeStruct((_N, _AF), jnp.float32),
        compiler_params=pltpu.CompilerParams(dimension_semantics=("arbitrary",)),
    )(x, ns, sum2, sq2, g2, b2)


def _pass3_pool(x, ns, sum2, sq2, g2, b2, cnt):
    cpb = _NCRYS // _GRID

    def body(x_ref, ns_ref, s2_ref, q2_ref, g2_ref, b2_ref, cnt_ref, out_ref):
        xn = _bn2_apply(x_ref, ns_ref, s2_ref, q2_ref, g2_ref, b2_ref)
        rows = lax.broadcasted_iota(jnp.int32, (cpb, _BN), 0)
        cols = lax.broadcasted_iota(jnp.int32, (cpb, _BN), 1)
        sel = (cols // _APC == rows).astype(jnp.float32)
        pooled = jnp.dot(sel, xn, preferred_element_type=jnp.float32)
        out_ref[0] = pooled / cnt_ref[0]

    return pl.pallas_call(
        body,
        grid=(_GRID,),
        in_specs=[
            pl.BlockSpec((_BN, _AF), lambda i: (i, 0)),
            pl.BlockSpec((_BN, _AF), lambda i: (i, 0)),
            pl.BlockSpec((1, _AF), lambda i: (0, 0)),
            pl.BlockSpec((1, _AF), lambda i: (0, 0)),
            pl.BlockSpec((1, _AF), lambda i: (0, 0)),
            pl.BlockSpec((1, _AF), lambda i: (0, 0)),
            pl.BlockSpec((1, cpb, 1), lambda i: (i, 0, 0)),
        ],
        out_specs=pl.BlockSpec((1, cpb, _AF), lambda i: (i, 0, 0)),
        out_shape=jax.ShapeDtypeStruct((_GRID, cpb, _AF), jnp.float32),
        compiler_params=pltpu.CompilerParams(dimension_semantics=("arbitrary",)),
    )(x, ns, sum2, sq2, g2, b2, cnt)


def _embed(afp, wep):
    kp = afp.shape[1]

    def body(a_ref, w_ref, out_ref):
        out_ref[...] = jnp.dot(a_ref[...], w_ref[...],
                               preferred_element_type=jnp.float32)

    return pl.pallas_call(
        body,
        grid=(_GRID,),
        in_specs=[
            pl.BlockSpec((_BN, kp), lambda i: (i, 0)),
            pl.BlockSpec((kp, _AF), lambda i: (0, 0)),
        ],
        out_specs=pl.BlockSpec((_BN, _AF), lambda i: (i, 0)),
        out_shape=jax.ShapeDtypeStruct((_N, _AF), jnp.float32),
        compiler_params=pltpu.CompilerParams(dimension_semantics=("arbitrary",)),
    )(afp, wep)


def kernel(atom_fea, nbr_fea, nbr_fea_idx, crystal_atom_idx, W_embed, fc_W,
           fc_b, bn1_g, bn1_b, bn2_g, bn2_b):
    idx2d = jnp.transpose(nbr_fea_idx.astype(jnp.int32)).reshape(_NCHUNK, _CH)
    nbrT = jnp.transpose(nbr_fea.astype(jnp.float32))[:, :, None]
    filt = jnp.concatenate(
        [jnp.arange(_NBF, dtype=jnp.float32) * 0.2,
         jnp.zeros((_NBF_P - _NBF,), jnp.float32)]).reshape(1, _NBF_P)
    kpad = 96
    afp = jnp.pad(atom_fea.astype(jnp.float32), ((0, 0), (0, kpad - atom_fea.shape[1])))
    wep = jnp.pad(W_embed.astype(jnp.float32), ((0, kpad - W_embed.shape[0]), (0, 0)))
    cnt = crystal_atom_idx.astype(jnp.float32).reshape(
        _GRID, _NCRYS // _GRID, 1)

    x = _embed(afp, wep)
    out = None
    for i in range(_NCONV):
        ws = fc_W[i, :_AF]
        wn = fc_W[i, _AF:2 * _AF].astype(jnp.bfloat16)
        wg = jnp.pad(fc_W[i, 2 * _AF:],
                     ((0, _NBF_P - _NBF), (0, 0))).astype(jnp.bfloat16)
        fcb = fc_b[i][None]
        g1 = bn1_g[i][None]
        b1 = bn1_b[i][None]
        g2 = bn2_g[i][None]
        b2 = bn2_b[i][None]

        xg = _sc_gather(x, idx2d).reshape(_M, _N, _AF)
        sum1, sq1 = _pass1(xg, x, nbrT, ws, wn, wg, fcb, filt)
        ns, sum2, sq2 = _pass2(xg, x, nbrT, ws, wn, wg, fcb, filt,
                               sum1, sq1, g1, b1)
        if i < _NCONV - 1:
            x = _pass3(x, ns, sum2, sq2, g2, b2)
        else:
            out = _pass3_pool(x, ns, sum2, sq2, g2, b2, cnt)
    return out.reshape(_NCRYS, _AF)


# speedup vs baseline: 1.6813x; 1.6813x over previous
import functools

import jax
import jax.numpy as jnp
from jax import lax
from jax.experimental import pallas as pl
from jax.experimental.pallas import tpu as pltpu
from jax.experimental.pallas import tpu_sc as plsc

_N = 10000
_M = 16
_AF = 128
_NCONV = 3
_NCRYS = 200
_APC = _N // _NCRYS
_NBF = 41
_NBF_P = 48
_INV_VAR = 1.0 / (0.2 * 0.2)

_E = _N * _M

_SC_NC = 2
_SC_NS = 16
_NW = _SC_NC * _SC_NS
_RPW = _E // _NW
_CH1 = 120
_NC1 = 41
_CH2 = _RPW - _NC1 * _CH1

_BN = 1000
_GRID = _N // _BN


def _sc_gather(x, idx):
    dt = x.dtype
    width = x.shape[1]
    mesh = plsc.VectorSubcoreMesh(core_axis_name="c", subcore_axis_name="s")

    @functools.partial(
        pl.kernel,
        mesh=mesh,
        out_type=jax.ShapeDtypeStruct((_E, width), dt),
        scratch_types=[
            pltpu.VMEM((_RPW,), jnp.int32),
            pltpu.VMEM((_CH1, width), dt),
            pltpu.VMEM((_CH1, width), dt),
            pltpu.SemaphoreType.DMA,
            pltpu.SemaphoreType.DMA,
        ],
    )
    def gath(x_hbm, idx_hbm, out_hbm, idx_v, rows0, rows1, sem0, sem1):
        wid = lax.axis_index("s") * _SC_NC + lax.axis_index("c")
        base = wid * _RPW
        pltpu.sync_copy(idx_hbm.at[pl.ds(base, _RPW)], idx_v)
        bufs = (rows0, rows1)
        sems = (sem0, sem1)
        pltpu.async_copy(x_hbm.at[idx_v.at[pl.ds(0, _CH1)]], rows0, sem0)

        def body(k, carry):
            for b in range(2):
                c = k * 2 + b
                nb = 1 - b

                @pl.when(c + 1 < _NC1)
                def _pref():
                    pltpu.async_copy(
                        x_hbm.at[idx_v.at[pl.ds((c + 1) * _CH1, _CH1)]],
                        bufs[nb], sems[nb])

                pltpu.make_async_copy(
                    x_hbm.at[idx_v.at[pl.ds(c * _CH1, _CH1)]],
                    bufs[b], sems[b]).wait()
                pltpu.sync_copy(bufs[b],
                                out_hbm.at[pl.ds(base + c * _CH1, _CH1)])
            return carry

        lax.fori_loop(0, _NC1 // 2, body, 0)
        tail0 = _NC1 * _CH1
        pltpu.async_copy(x_hbm.at[idx_v.at[pl.ds(tail0, _CH2)]],
                         rows1.at[pl.ds(0, _CH2)], sem1)
        pltpu.make_async_copy(
            x_hbm.at[idx_v.at[pl.ds((_NC1 - 1) * _CH1, _CH1)]],
            rows0, sem0).wait()
        pltpu.sync_copy(rows0,
                        out_hbm.at[pl.ds(base + (_NC1 - 1) * _CH1, _CH1)])
        pltpu.make_async_copy(x_hbm.at[idx_v.at[pl.ds(tail0, _CH2)]],
                              rows1.at[pl.ds(0, _CH2)], sem1).wait()
        pltpu.sync_copy(rows1.at[pl.ds(0, _CH2)],
                        out_hbm.at[pl.ds(base + tail0, _CH2)])

    return gath(x, idx)


def _softplus(v):
    return jnp.maximum(v, 0.0) + jnp.log(1.0 + jnp.exp(-jnp.abs(v)))


def _wide_specs():
    return [
        pl.BlockSpec((_M, _BN, _AF), lambda i: (0, i, 0)),
        pl.BlockSpec((_BN, _AF), lambda i: (i, 0)),
        pl.BlockSpec((_M, _BN, 1), lambda i: (0, i, 0)),
        pl.BlockSpec((_AF, 2 * _AF), lambda i: (0, 0)),
        pl.BlockSpec((_AF, 2 * _AF), lambda i: (0, 0)),
        pl.BlockSpec((_NBF_P, 2 * _AF), lambda i: (0, 0)),
        pl.BlockSpec((1, 2 * _AF), lambda i: (0, 0)),
        pl.BlockSpec((1, _NBF_P), lambda i: (0, 0)),
    ]


def _gated_m(m, xg_ref, nbr_ref, wn_ref, wg_ref, filt_ref, s):
    g = jnp.exp(-jnp.square(nbr_ref[m] - filt_ref[...]) * _INV_VAR)
    um = jnp.dot(xg_ref[m].astype(jnp.bfloat16), wn_ref[...],
                 preferred_element_type=jnp.float32)
    gm = jnp.dot(g.astype(wg_ref.dtype), wg_ref[...],
                 preferred_element_type=jnp.float32)
    return um + gm + s


def _pass1(xg, x, nbrT, ws, wn, wg, fcb, filt):
    def body(xg_ref, x_ref, nbr_ref, ws_ref, wn_ref, wg_ref,
             fcb_ref, filt_ref, sum_ref, sq_ref):
        s = jnp.dot(x_ref[...], ws_ref[...],
                    preferred_element_type=jnp.float32) + fcb_ref[...]
        tot = jnp.zeros((1, 2 * _AF), jnp.float32)
        tot2 = jnp.zeros((1, 2 * _AF), jnp.float32)
        for m in range(_M):
            gated = _gated_m(m, xg_ref, nbr_ref, wn_ref, wg_ref, filt_ref, s)
            tot = tot + jnp.sum(gated, axis=0, keepdims=True)
            tot2 = tot2 + jnp.sum(gated * gated, axis=0, keepdims=True)

        @pl.when(pl.program_id(0) == 0)
        def _init():
            sum_ref[...] = jnp.zeros_like(sum_ref)
            sq_ref[...] = jnp.zeros_like(sq_ref)

        sum_ref[...] += tot
        sq_ref[...] += tot2

    return pl.pallas_call(
        body,
        grid=(_GRID,),
        in_specs=_wide_specs(),
        out_specs=[
            pl.BlockSpec((1, 2 * _AF), lambda i: (0, 0)),
            pl.BlockSpec((1, 2 * _AF), lambda i: (0, 0)),
        ],
        out_shape=[
            jax.ShapeDtypeStruct((1, 2 * _AF), jnp.float32),
            jax.ShapeDtypeStruct((1, 2 * _AF), jnp.float32),
        ],
        compiler_params=pltpu.CompilerParams(dimension_semantics=("arbitrary",)),
    )(xg, x, nbrT, ws, wn, wg, fcb, filt)


def _pass2(xg, x, nbrT, ws, wn, wg, fcb, filt, sum1, sq1, g1, b1):
    def body(xg_ref, x_ref, nbr_ref, ws_ref, wn_ref, wg_ref,
             fcb_ref, filt_ref, s1_ref, q1_ref, g1_ref, b1_ref,
             ns_ref, sum_ref, sq_ref):
        cnt = jnp.float32(_E)
        mu = s1_ref[...] / cnt
        var = q1_ref[...] / cnt - mu * mu
        inv = lax.rsqrt(var + 1e-5)
        scale = g1_ref[...] * inv
        shift = b1_ref[...] - mu * scale
        s = jnp.dot(x_ref[...], ws_ref[...],
                    preferred_element_type=jnp.float32) + fcb_ref[...]
        acc = jnp.zeros((_BN, _AF), jnp.float32)
        for m in range(_M):
            gated = _gated_m(m, xg_ref, nbr_ref, wn_ref, wg_ref, filt_ref, s)
            gh = gated * scale + shift
            f = 1.0 / (1.0 + jnp.exp(-gh[:, :_AF]))
            c = _softplus(gh[:, _AF:])
            acc = acc + f * c
        ns_ref[...] = acc

        @pl.when(pl.program_id(0) == 0)
        def _init():
            sum_ref[...] = jnp.zeros_like(sum_ref)
            sq_ref[...] = jnp.zeros_like(sq_ref)

        sum_ref[...] += jnp.sum(acc, axis=0, keepdims=True)
        sq_ref[...] += jnp.sum(acc * acc, axis=0, keepdims=True)

    return pl.pallas_call(
        body,
        grid=(_GRID,),
        in_specs=_wide_specs() + [
            pl.BlockSpec((1, 2 * _AF), lambda i: (0, 0)),
            pl.BlockSpec((1, 2 * _AF), lambda i: (0, 0)),
            pl.BlockSpec((1, 2 * _AF), lambda i: (0, 0)),
            pl.BlockSpec((1, 2 * _AF), lambda i: (0, 0)),
        ],
        out_specs=[
            pl.BlockSpec((_BN, _AF), lambda i: (i, 0)),
            pl.BlockSpec((1, _AF), lambda i: (0, 0)),
            pl.BlockSpec((1, _AF), lambda i: (0, 0)),
        ],
        out_shape=[
            jax.ShapeDtypeStruct((_N, _AF), jnp.float32),
            jax.ShapeDtypeStruct((1, _AF), jnp.float32),
            jax.ShapeDtypeStruct((1, _AF), jnp.float32),
        ],
        compiler_params=pltpu.CompilerParams(dimension_semantics=("arbitrary",)),
    )(xg, x, nbrT, ws, wn, wg, fcb, filt, sum1, sq1, g1, b1)


def _bn2_apply(x_ref, ns_ref, s2_ref, q2_ref, g2_ref, b2_ref):
    cnt = jnp.float32(_N)
    mu = s2_ref[...] / cnt
    var = q2_ref[...] / cnt - mu * mu
    inv = lax.rsqrt(var + 1e-5)
    scale = g2_ref[...] * inv
    shift = b2_ref[...] - mu * scale
    return _softplus(x_ref[...] + ns_ref[...] * scale + shift)


def _pass3(x, ns, sum2, sq2, g2, b2):
    def body(x_ref, ns_ref, s2_ref, q2_ref, g2_ref, b2_ref, out_ref):
        out_ref[...] = _bn2_apply(x_ref, ns_ref, s2_ref, q2_ref, g2_ref, b2_ref)

    return pl.pallas_call(
        body,
        grid=(_GRID,),
        in_specs=[
            pl.BlockSpec((_BN, _AF), lambda i: (i, 0)),
            pl.BlockSpec((_BN, _AF), lambda i: (i, 0)),
            pl.BlockSpec((1, _AF), lambda i: (0, 0)),
            pl.BlockSpec((1, _AF), lambda i: (0, 0)),
            pl.BlockSpec((1, _AF), lambda i: (0, 0)),
            pl.BlockSpec((1, _AF), lambda i: (0, 0)),
        ],
        out_specs=pl.BlockSpec((_BN, _AF), lambda i: (i, 0)),
        out_shape=jax.ShapeDtypeStruct((_N, _AF), jnp.float32),
        compiler_params=pltpu.CompilerParams(dimension_semantics=("arbitrary",)),
    )(x, ns, sum2, sq2, g2, b2)


def _pass3_pool(x, ns, sum2, sq2, g2, b2, cnt):
    cpb = _NCRYS // _GRID

    def body(x_ref, ns_ref, s2_ref, q2_ref, g2_ref, b2_ref, cnt_ref, out_ref):
        xn = _bn2_apply(x_ref, ns_ref, s2_ref, q2_ref, g2_ref, b2_ref)
        rows = lax.broadcasted_iota(jnp.int32, (cpb, _BN), 0)
        cols = lax.broadcasted_iota(jnp.int32, (cpb, _BN), 1)
        sel = (cols // _APC == rows).astype(jnp.float32)
        pooled = jnp.dot(sel, xn, preferred_element_type=jnp.float32)
        out_ref[0] = pooled / cnt_ref[0]

    return pl.pallas_call(
        body,
        grid=(_GRID,),
        in_specs=[
            pl.BlockSpec((_BN, _AF), lambda i: (i, 0)),
            pl.BlockSpec((_BN, _AF), lambda i: (i, 0)),
            pl.BlockSpec((1, _AF), lambda i: (0, 0)),
            pl.BlockSpec((1, _AF), lambda i: (0, 0)),
            pl.BlockSpec((1, _AF), lambda i: (0, 0)),
            pl.BlockSpec((1, _AF), lambda i: (0, 0)),
            pl.BlockSpec((1, cpb, 1), lambda i: (i, 0, 0)),
        ],
        out_specs=pl.BlockSpec((1, cpb, _AF), lambda i: (i, 0, 0)),
        out_shape=jax.ShapeDtypeStruct((_GRID, cpb, _AF), jnp.float32),
        compiler_params=pltpu.CompilerParams(dimension_semantics=("arbitrary",)),
    )(x, ns, sum2, sq2, g2, b2, cnt)


def _embed(afp, wep):
    kp = afp.shape[1]

    def body(a_ref, w_ref, out_ref):
        out_ref[...] = jnp.dot(a_ref[...], w_ref[...],
                               preferred_element_type=jnp.float32)

    return pl.pallas_call(
        body,
        grid=(_GRID,),
        in_specs=[
            pl.BlockSpec((_BN, kp), lambda i: (i, 0)),
            pl.BlockSpec((kp, _AF), lambda i: (0, 0)),
        ],
        out_specs=pl.BlockSpec((_BN, _AF), lambda i: (i, 0)),
        out_shape=jax.ShapeDtypeStruct((_N, _AF), jnp.float32),
        compiler_params=pltpu.CompilerParams(dimension_semantics=("arbitrary",)),
    )(afp, wep)


def kernel(atom_fea, nbr_fea, nbr_fea_idx, crystal_atom_idx, W_embed, fc_W,
           fc_b, bn1_g, bn1_b, bn2_g, bn2_b):
    idxf = jnp.transpose(nbr_fea_idx.astype(jnp.int32)).reshape(_E)
    nbrT = jnp.transpose(nbr_fea.astype(jnp.float32))[:, :, None]
    filt = jnp.concatenate(
        [jnp.arange(_NBF, dtype=jnp.float32) * 0.2,
         jnp.zeros((_NBF_P - _NBF,), jnp.float32)]).reshape(1, _NBF_P)
    kpad = 96
    afp = jnp.pad(atom_fea.astype(jnp.float32), ((0, 0), (0, kpad - atom_fea.shape[1])))
    wep = jnp.pad(W_embed.astype(jnp.float32), ((0, kpad - W_embed.shape[0]), (0, 0)))
    cnt = crystal_atom_idx.astype(jnp.float32).reshape(
        _GRID, _NCRYS // _GRID, 1)

    x = _embed(afp, wep)
    out = None
    for i in range(_NCONV):
        ws = fc_W[i, :_AF]
        wn = fc_W[i, _AF:2 * _AF].astype(jnp.bfloat16)
        wg = jnp.pad(fc_W[i, 2 * _AF:],
                     ((0, _NBF_P - _NBF), (0, 0))).astype(jnp.bfloat16)
        fcb = fc_b[i][None]
        g1 = bn1_g[i][None]
        b1 = bn1_b[i][None]
        g2 = bn2_g[i][None]
        b2 = bn2_b[i][None]

        xg = _sc_gather(x, idxf).reshape(_M, _N, _AF)
        sum1, sq1 = _pass1(xg, x, nbrT, ws, wn, wg, fcb, filt)
        ns, sum2, sq2 = _pass2(xg, x, nbrT, ws, wn, wg, fcb, filt,
                               sum1, sq1, g1, b1)
        if i < _NCONV - 1:
            x = _pass3(x, ns, sum2, sq2, g2, b2)
        else:
            out = _pass3_pool(x, ns, sum2, sq2, g2, b2, cnt)
    return out.reshape(_NCRYS, _AF)


# speedup vs baseline: 1.7645x; 1.0495x over previous
import functools

import jax
import jax.numpy as jnp
from jax import lax
from jax.experimental import pallas as pl
from jax.experimental.pallas import tpu as pltpu
from jax.experimental.pallas import tpu_sc as plsc

_N = 10000
_M = 16
_AF = 128
_NCONV = 3
_NCRYS = 200
_APC = _N // _NCRYS
_NBF = 41
_NBF_P = 48
_INV_VAR = 1.0 / (0.2 * 0.2)

_E = _N * _M

_SC_NC = 2
_SC_NS = 16
_NW = _SC_NC * _SC_NS
_RPW = _E // _NW
_CH1 = 120
_NC1 = 41
_CH2 = _RPW - _NC1 * _CH1

_BN = 1000
_GRID = _N // _BN


def _sc_gather(x, idx):
    dt = x.dtype
    width = x.shape[1]
    mesh = plsc.VectorSubcoreMesh(core_axis_name="c", subcore_axis_name="s")

    @functools.partial(
        pl.kernel,
        mesh=mesh,
        out_type=jax.ShapeDtypeStruct((_E, width), dt),
        scratch_types=[
            pltpu.VMEM((_RPW,), jnp.int32),
            pltpu.VMEM((_CH1, width), dt),
            pltpu.VMEM((_CH1, width), dt),
            pltpu.SemaphoreType.DMA,
            pltpu.SemaphoreType.DMA,
        ],
    )
    def gath(x_hbm, idx_hbm, out_hbm, idx_v, rows0, rows1, sem0, sem1):
        wid = lax.axis_index("s") * _SC_NC + lax.axis_index("c")
        base = wid * _RPW
        pltpu.sync_copy(idx_hbm.at[pl.ds(base, _RPW)], idx_v)
        bufs = (rows0, rows1)
        sems = (sem0, sem1)
        pltpu.async_copy(x_hbm.at[idx_v.at[pl.ds(0, _CH1)]], rows0, sem0)

        def body(k, carry):
            for b in range(2):
                c = k * 2 + b
                nb = 1 - b

                @pl.when(c + 1 < _NC1)
                def _pref():
                    pltpu.async_copy(
                        x_hbm.at[idx_v.at[pl.ds((c + 1) * _CH1, _CH1)]],
                        bufs[nb], sems[nb])

                pltpu.make_async_copy(
                    x_hbm.at[idx_v.at[pl.ds(c * _CH1, _CH1)]],
                    bufs[b], sems[b]).wait()
                pltpu.sync_copy(bufs[b],
                                out_hbm.at[pl.ds(base + c * _CH1, _CH1)])
            return carry

        lax.fori_loop(0, _NC1 // 2, body, 0)
        tail0 = _NC1 * _CH1
        pltpu.async_copy(x_hbm.at[idx_v.at[pl.ds(tail0, _CH2)]],
                         rows1.at[pl.ds(0, _CH2)], sem1)
        pltpu.make_async_copy(
            x_hbm.at[idx_v.at[pl.ds((_NC1 - 1) * _CH1, _CH1)]],
            rows0, sem0).wait()
        pltpu.sync_copy(rows0,
                        out_hbm.at[pl.ds(base + (_NC1 - 1) * _CH1, _CH1)])
        pltpu.make_async_copy(x_hbm.at[idx_v.at[pl.ds(tail0, _CH2)]],
                              rows1.at[pl.ds(0, _CH2)], sem1).wait()
        pltpu.sync_copy(rows1.at[pl.ds(0, _CH2)],
                        out_hbm.at[pl.ds(base + tail0, _CH2)])

    return gath(x, idx)


def _softplus(v):
    return jnp.maximum(v, 0.0) + jnp.log(1.0 + jnp.exp(-jnp.abs(v)))


def _wide_specs():
    return [
        pl.BlockSpec((_M, _BN, _AF), lambda i: (0, i, 0)),
        pl.BlockSpec((_BN, _AF), lambda i: (i, 0)),
        pl.BlockSpec((_M, _BN, 1), lambda i: (0, i, 0)),
        pl.BlockSpec((_AF, 2 * _AF), lambda i: (0, 0)),
        pl.BlockSpec((_AF, 2 * _AF), lambda i: (0, 0)),
        pl.BlockSpec((_NBF_P, 2 * _AF), lambda i: (0, 0)),
        pl.BlockSpec((1, 2 * _AF), lambda i: (0, 0)),
        pl.BlockSpec((1, _NBF_P), lambda i: (0, 0)),
    ]


def _gated_m(m, xg_ref, nbr_ref, wn_ref, wg_ref, filt_ref, s):
    g = jnp.exp(-jnp.square(nbr_ref[m] - filt_ref[...]) * _INV_VAR)
    um = jnp.dot(xg_ref[m].astype(jnp.bfloat16), wn_ref[...],
                 preferred_element_type=jnp.float32)
    gm = jnp.dot(g.astype(wg_ref.dtype), wg_ref[...],
                 preferred_element_type=jnp.float32)
    return um + gm + s


def _pass1(xg, x, nbrT, ws, wn, wg, fcb, filt):
    def body(xg_ref, x_ref, nbr_ref, ws_ref, wn_ref, wg_ref,
             fcb_ref, filt_ref, sum_ref, sq_ref):
        s = jnp.dot(x_ref[...], ws_ref[...],
                    preferred_element_type=jnp.float32) + fcb_ref[...]
        tot = jnp.zeros((1, 2 * _AF), jnp.float32)
        tot2 = jnp.zeros((1, 2 * _AF), jnp.float32)
        for m in range(_M):
            gated = _gated_m(m, xg_ref, nbr_ref, wn_ref, wg_ref, filt_ref, s)
            tot = tot + jnp.sum(gated, axis=0, keepdims=True)
            tot2 = tot2 + jnp.sum(gated * gated, axis=0, keepdims=True)

        @pl.when(pl.program_id(0) == 0)
        def _init():
            sum_ref[...] = jnp.zeros_like(sum_ref)
            sq_ref[...] = jnp.zeros_like(sq_ref)

        sum_ref[...] += tot
        sq_ref[...] += tot2

    return pl.pallas_call(
        body,
        grid=(_GRID,),
        in_specs=_wide_specs(),
        out_specs=[
            pl.BlockSpec((1, 2 * _AF), lambda i: (0, 0)),
            pl.BlockSpec((1, 2 * _AF), lambda i: (0, 0)),
        ],
        out_shape=[
            jax.ShapeDtypeStruct((1, 2 * _AF), jnp.float32),
            jax.ShapeDtypeStruct((1, 2 * _AF), jnp.float32),
        ],
        compiler_params=pltpu.CompilerParams(dimension_semantics=("arbitrary",)),
    )(xg, x, nbrT, ws, wn, wg, fcb, filt)


def _pass2(xg, x, nbrT, ws, wn, wg, fcb, filt, sum1, sq1, g1, b1):
    def body(xg_ref, x_ref, nbr_ref, ws_ref, wn_ref, wg_ref,
             fcb_ref, filt_ref, s1_ref, q1_ref, g1_ref, b1_ref,
             ns_ref, sum_ref, sq_ref):
        cnt = jnp.float32(_E)
        mu = s1_ref[...] / cnt
        var = q1_ref[...] / cnt - mu * mu
        inv = lax.rsqrt(var + 1e-5)
        scale = g1_ref[...] * inv
        shift = b1_ref[...] - mu * scale
        wn_s = (wn_ref[...].astype(jnp.float32) * scale).astype(jnp.bfloat16)
        wg_s = (wg_ref[...].astype(jnp.float32) * scale).astype(jnp.bfloat16)
        s = jnp.dot(x_ref[...], ws_ref[...],
                    preferred_element_type=jnp.float32) + fcb_ref[...]
        s = s * scale + shift
        acc = jnp.zeros((_BN, _AF), jnp.float32)
        for m in range(_M):
            g = jnp.exp(-jnp.square(nbr_ref[m] - filt_ref[...]) * _INV_VAR)
            gh = (jnp.dot(xg_ref[m].astype(jnp.bfloat16), wn_s,
                          preferred_element_type=jnp.float32)
                  + jnp.dot(g.astype(jnp.bfloat16), wg_s,
                            preferred_element_type=jnp.float32)
                  + s)
            f = 0.5 * jnp.tanh(0.5 * gh[:, :_AF]) + 0.5
            c = _softplus(gh[:, _AF:])
            acc = acc + f * c
        ns_ref[...] = acc

        @pl.when(pl.program_id(0) == 0)
        def _init():
            sum_ref[...] = jnp.zeros_like(sum_ref)
            sq_ref[...] = jnp.zeros_like(sq_ref)

        sum_ref[...] += jnp.sum(acc, axis=0, keepdims=True)
        sq_ref[...] += jnp.sum(acc * acc, axis=0, keepdims=True)

    return pl.pallas_call(
        body,
        grid=(_GRID,),
        in_specs=_wide_specs() + [
            pl.BlockSpec((1, 2 * _AF), lambda i: (0, 0)),
            pl.BlockSpec((1, 2 * _AF), lambda i: (0, 0)),
            pl.BlockSpec((1, 2 * _AF), lambda i: (0, 0)),
            pl.BlockSpec((1, 2 * _AF), lambda i: (0, 0)),
        ],
        out_specs=[
            pl.BlockSpec((_BN, _AF), lambda i: (i, 0)),
            pl.BlockSpec((1, _AF), lambda i: (0, 0)),
            pl.BlockSpec((1, _AF), lambda i: (0, 0)),
        ],
        out_shape=[
            jax.ShapeDtypeStruct((_N, _AF), jnp.float32),
            jax.ShapeDtypeStruct((1, _AF), jnp.float32),
            jax.ShapeDtypeStruct((1, _AF), jnp.float32),
        ],
        compiler_params=pltpu.CompilerParams(dimension_semantics=("arbitrary",)),
    )(xg, x, nbrT, ws, wn, wg, fcb, filt, sum1, sq1, g1, b1)


def _bn2_apply(x_ref, ns_ref, s2_ref, q2_ref, g2_ref, b2_ref):
    cnt = jnp.float32(_N)
    mu = s2_ref[...] / cnt
    var = q2_ref[...] / cnt - mu * mu
    inv = lax.rsqrt(var + 1e-5)
    scale = g2_ref[...] * inv
    shift = b2_ref[...] - mu * scale
    return _softplus(x_ref[...] + ns_ref[...] * scale + shift)


def _pass3(x, ns, sum2, sq2, g2, b2):
    def body(x_ref, ns_ref, s2_ref, q2_ref, g2_ref, b2_ref, out_ref):
        out_ref[...] = _bn2_apply(x_ref, ns_ref, s2_ref, q2_ref, g2_ref, b2_ref)

    return pl.pallas_call(
        body,
        grid=(_GRID,),
        in_specs=[
            pl.BlockSpec((_BN, _AF), lambda i: (i, 0)),
            pl.BlockSpec((_BN, _AF), lambda i: (i, 0)),
            pl.BlockSpec((1, _AF), lambda i: (0, 0)),
            pl.BlockSpec((1, _AF), lambda i: (0, 0)),
            pl.BlockSpec((1, _AF), lambda i: (0, 0)),
            pl.BlockSpec((1, _AF), lambda i: (0, 0)),
        ],
        out_specs=pl.BlockSpec((_BN, _AF), lambda i: (i, 0)),
        out_shape=jax.ShapeDtypeStruct((_N, _AF), jnp.float32),
        compiler_params=pltpu.CompilerParams(dimension_semantics=("arbitrary",)),
    )(x, ns, sum2, sq2, g2, b2)


def _pass3_pool(x, ns, sum2, sq2, g2, b2, cnt):
    cpb = _NCRYS // _GRID

    def body(x_ref, ns_ref, s2_ref, q2_ref, g2_ref, b2_ref, cnt_ref, out_ref):
        xn = _bn2_apply(x_ref, ns_ref, s2_ref, q2_ref, g2_ref, b2_ref)
        rows = lax.broadcasted_iota(jnp.int32, (cpb, _BN), 0)
        cols = lax.broadcasted_iota(jnp.int32, (cpb, _BN), 1)
        sel = (cols // _APC == rows).astype(jnp.float32)
        pooled = jnp.dot(sel, xn, preferred_element_type=jnp.float32)
        out_ref[0] = pooled / cnt_ref[0]

    return pl.pallas_call(
        body,
        grid=(_GRID,),
        in_specs=[
            pl.BlockSpec((_BN, _AF), lambda i: (i, 0)),
            pl.BlockSpec((_BN, _AF), lambda i: (i, 0)),
            pl.BlockSpec((1, _AF), lambda i: (0, 0)),
            pl.BlockSpec((1, _AF), lambda i: (0, 0)),
            pl.BlockSpec((1, _AF), lambda i: (0, 0)),
            pl.BlockSpec((1, _AF), lambda i: (0, 0)),
            pl.BlockSpec((1, cpb, 1), lambda i: (i, 0, 0)),
        ],
        out_specs=pl.BlockSpec((1, cpb, _AF), lambda i: (i, 0, 0)),
        out_shape=jax.ShapeDtypeStruct((_GRID, cpb, _AF), jnp.float32),
        compiler_params=pltpu.CompilerParams(dimension_semantics=("arbitrary",)),
    )(x, ns, sum2, sq2, g2, b2, cnt)


def _embed(afp, wep):
    kp = afp.shape[1]

    def body(a_ref, w_ref, out_ref):
        out_ref[...] = jnp.dot(a_ref[...], w_ref[...],
                               preferred_element_type=jnp.float32)

    return pl.pallas_call(
        body,
        grid=(_GRID,),
        in_specs=[
            pl.BlockSpec((_BN, kp), lambda i: (i, 0)),
            pl.BlockSpec((kp, _AF), lambda i: (0, 0)),
        ],
        out_specs=pl.BlockSpec((_BN, _AF), lambda i: (i, 0)),
        out_shape=jax.ShapeDtypeStruct((_N, _AF), jnp.float32),
        compiler_params=pltpu.CompilerParams(dimension_semantics=("arbitrary",)),
    )(afp, wep)


def kernel(atom_fea, nbr_fea, nbr_fea_idx, crystal_atom_idx, W_embed, fc_W,
           fc_b, bn1_g, bn1_b, bn2_g, bn2_b):
    idxf = jnp.transpose(nbr_fea_idx.astype(jnp.int32)).reshape(_E)
    nbrT = jnp.transpose(nbr_fea.astype(jnp.float32))[:, :, None]
    filt = jnp.concatenate(
        [jnp.arange(_NBF, dtype=jnp.float32) * 0.2,
         jnp.zeros((_NBF_P - _NBF,), jnp.float32)]).reshape(1, _NBF_P)
    kpad = 96
    afp = jnp.pad(atom_fea.astype(jnp.float32), ((0, 0), (0, kpad - atom_fea.shape[1])))
    wep = jnp.pad(W_embed.astype(jnp.float32), ((0, kpad - W_embed.shape[0]), (0, 0)))
    cnt = crystal_atom_idx.astype(jnp.float32).reshape(
        _GRID, _NCRYS // _GRID, 1)

    x = _embed(afp, wep)
    out = None
    for i in range(_NCONV):
        ws = fc_W[i, :_AF]
        wn = fc_W[i, _AF:2 * _AF].astype(jnp.bfloat16)
        wg = jnp.pad(fc_W[i, 2 * _AF:],
                     ((0, _NBF_P - _NBF), (0, 0))).astype(jnp.bfloat16)
        fcb = fc_b[i][None]
        g1 = bn1_g[i][None]
        b1 = bn1_b[i][None]
        g2 = bn2_g[i][None]
        b2 = bn2_b[i][None]

        xg = _sc_gather(x, idxf).reshape(_M, _N, _AF)
        sum1, sq1 = _pass1(xg, x, nbrT, ws, wn, wg, fcb, filt)
        ns, sum2, sq2 = _pass2(xg, x, nbrT, ws, wn, wg, fcb, filt,
                               sum1, sq1, g1, b1)
        if i < _NCONV - 1:
            x = _pass3(x, ns, sum2, sq2, g2, b2)
        else:
            out = _pass3_pool(x, ns, sum2, sq2, g2, b2, cnt)
    return out.reshape(_NCRYS, _AF)
